# baseline (speedup 1.0000x reference)
import functools

import jax
import jax.numpy as jnp
from jax import lax
from jax.experimental import pallas as pl
from jax.experimental.pallas import tpu as pltpu
from jax.experimental.pallas import tpu_sc as plsc

_NC, _NS, _L = 2, 16, 16
_NW = _NC * _NS
_CH = 32


@functools.lru_cache(maxsize=None)
def _make_sc_kernel(B, S, D):
    rows_w = S // _NW
    mesh = plsc.VectorSubcoreMesh(core_axis_name="c", subcore_axis_name="s")

    @functools.partial(
        pl.kernel,
        out_type=jax.ShapeDtypeStruct((B, S, D), jnp.float32),
        mesh=mesh,
        scratch_types=[
            pltpu.VMEM((_CH, D), jnp.float32),
            pltpu.VMEM((_CH, D), jnp.float32),
        ],
    )
    def sc_add(x_hbm, pe_hbm, out_hbm, pev, xv):
        wid = lax.axis_index("s") * _NC + lax.axis_index("c")
        base = wid * rows_w

        def chunk(ci, carry):
            s0 = base + ci * _CH
            pltpu.sync_copy(pe_hbm.at[pl.ds(s0, _CH)], pev)

            def per_batch(b, carry2):
                pltpu.sync_copy(x_hbm.at[b, pl.ds(s0, _CH)], xv)

                def per_row(r, carry3):
                    for j in range(D // _L):
                        sl = pl.ds(j * _L, _L)
                        xv[r, sl] = xv[r, sl] + pev[r, sl]
                    return carry3

                lax.fori_loop(0, _CH, per_row, 0)
                pltpu.sync_copy(xv, out_hbm.at[b, pl.ds(s0, _CH)])
                return carry2

            lax.fori_loop(0, B, per_batch, 0)
            return carry

        lax.fori_loop(0, rows_w // _CH, chunk, 0)

    return sc_add


def kernel(x, pe_weight):
    B, S, D = x.shape
    pe = pe_weight[:S]
    return _make_sc_kernel(B, S, D)(x, pe)


# speedup vs baseline: 1.2801x; 1.2801x over previous
import functools

import jax
import jax.numpy as jnp
from jax import lax
from jax.experimental import pallas as pl
from jax.experimental.pallas import tpu as pltpu
from jax.experimental.pallas import tpu_sc as plsc

_NC, _NS, _L = 2, 16, 16
_NW = _NC * _NS
_CH = 16


@functools.lru_cache(maxsize=None)
def _make_sc_kernel(B, S, D):
    rows_w = S // _NW
    nch = rows_w // _CH
    mesh = plsc.VectorSubcoreMesh(core_axis_name="c", subcore_axis_name="s")
    buf = pltpu.VMEM((_CH, D), jnp.float32)
    sem = pltpu.SemaphoreType.DMA

    @functools.partial(
        pl.kernel,
        out_type=jax.ShapeDtypeStruct((B, S, D), jnp.float32),
        mesh=mesh,
        scratch_types=[buf] * (2 + B) + [sem] * (2 + 2 * B),
    )
    def sc_add(x_hbm, pe_hbm, out_hbm, *scratch):
        pe_bufs = tuple(zip(scratch[:2], scratch[2 + B : 4 + B]))
        x_refs = scratch[2 : 2 + B]
        in_sems = scratch[4 + B : 4 + 2 * B]
        out_sems = scratch[4 + 2 * B : 4 + 3 * B]

        wid = lax.axis_index("s") * _NC + lax.axis_index("c")
        base = wid * rows_w

        pltpu.async_copy(pe_hbm.at[pl.ds(base, _CH)], pe_bufs[0][0], pe_bufs[0][1])
        for b in range(B):
            pltpu.async_copy(x_hbm.at[b, pl.ds(base, _CH)], x_refs[b], in_sems[b])

        def chunk_pair(ci2, carry):
            for cpar in (0, 1):
                ci = ci2 * 2 + cpar
                s0 = base + ci * _CH
                peb, pes = pe_bufs[cpar]
                pltpu.make_async_copy(pe_hbm.at[pl.ds(s0, _CH)], peb, pes).wait()

                @pl.when(ci + 1 < nch)
                def _():
                    nb, ns = pe_bufs[1 - cpar]
                    pltpu.async_copy(
                        pe_hbm.at[pl.ds(s0 + _CH, _CH)], nb, ns
                    )

                for b in range(B):
                    xb = x_refs[b]
                    pltpu.make_async_copy(
                        x_hbm.at[b, pl.ds(s0, _CH)], xb, in_sems[b]
                    ).wait()

                    @plsc.parallel_loop(0, _CH)
                    def _row(r):
                        for j in range(D // _L):
                            sl = pl.ds(j * _L, _L)
                            plsc.addupdate(xb.at[r, sl], peb[r, sl])

                    pltpu.async_copy(xb, out_hbm.at[b, pl.ds(s0, _CH)], out_sems[b])

                @pl.when(ci + 1 < nch)
                def _():
                    for b in range(B):
                        xb = x_refs[b]
                        pltpu.make_async_copy(
                            xb, out_hbm.at[b, pl.ds(s0, _CH)], out_sems[b]
                        ).wait()
                        pltpu.async_copy(
                            x_hbm.at[b, pl.ds(s0 + _CH, _CH)], xb, in_sems[b]
                        )

            return carry

        lax.fori_loop(0, nch // 2, chunk_pair, 0)

        last = base + (nch - 1) * _CH
        for b in range(B):
            pltpu.make_async_copy(
                x_refs[b], out_hbm.at[b, pl.ds(last, _CH)], out_sems[b]
            ).wait()

    return sc_add


def kernel(x, pe_weight):
    B, S, D = x.shape
    pe = pe_weight[:S]
    return _make_sc_kernel(B, S, D)(x, pe)
